# baseline (speedup 1.0000x reference)
import jax
import jax.numpy as jnp
from jax import lax
from jax.experimental import pallas as pl

DIM = 1024
QD = 128
NH = 8
K = 8
SIDE = 1024
SUB = QD // 2
BSZ = 2
SEQ = 4096
TOKENS = BSZ * SEQ
GROUP = TOKENS // NH
TILE = 512
TPG = GROUP // TILE
ROWS = TILE * NH


def _topk8_ax0(s, n):
    iota = lax.broadcasted_iota(jnp.int32, s.shape, 0)
    neg_inf = jnp.float32(-jnp.inf)
    vals, idxs = [], []
    cur = s
    for k in range(K):
        m = jnp.max(cur, axis=0, keepdims=True)
        am = jnp.argmax(cur, axis=0, keepdims=True).astype(jnp.int32)
        vals.append(m)
        idxs.append(am)
        if k + 1 < K:
            cur = jnp.where(iota == am, neg_inf, cur)
    return jnp.concatenate(vals, axis=0), jnp.concatenate(idxs, axis=0)


def _router_kernel(xt_ref, wt_ref, b_ref, k1_ref, k2_ref, sv_ref, si_ref):
    q = jnp.dot(wt_ref[...], xt_ref[...], preferred_element_type=jnp.float32)
    q = q + b_ref[...]
    q1 = jnp.concatenate(
        [q[h * QD:h * QD + SUB, :] for h in range(NH)], axis=1)
    q2 = jnp.concatenate(
        [q[h * QD + SUB:(h + 1) * QD, :] for h in range(NH)], axis=1)
    s1 = jnp.dot(k1_ref[0], q1, preferred_element_type=jnp.float32)
    s2 = jnp.dot(k2_ref[0], q2, preferred_element_type=jnp.float32)
    v1, i1 = _topk8_ax0(s1, SIDE)
    v2, i2 = _topk8_ax0(s2, SIDE)
    c = jnp.concatenate([v1[j:j + 1, :] + v2 for j in range(K)], axis=0)
    cv, ci = _topk8_ax0(c, K * K)
    sub1 = ci // K
    sub2 = ci - sub1 * K
    a1 = jnp.zeros_like(ci)
    a2 = jnp.zeros_like(ci)
    for j in range(K):
        a1 = jnp.where(sub1 == j, i1[j:j + 1, :], a1)
        a2 = jnp.where(sub2 == j, i2[j:j + 1, :], a2)
    eid = a1 * SIDE + a2
    for h in range(NH):
        sv_ref[0, h * K:(h + 1) * K, :] = cv[:, h * TILE:(h + 1) * TILE]
        si_ref[0, h * K:(h + 1) * K, :] = eid[:, h * TILE:(h + 1) * TILE]


def kernel(x, W, b, keys):
    xt = x.reshape(TOKENS, DIM).T
    wt = W.T
    b2 = b.reshape(NH * QD, 1)
    k1 = keys[:, 0]
    k2 = keys[:, 1]
    grid = (TOKENS // TILE,)
    sv, si = pl.pallas_call(
        _router_kernel,
        grid=grid,
        in_specs=[
            pl.BlockSpec((DIM, TILE), lambda i: (0, i)),
            pl.BlockSpec((NH * QD, DIM), lambda i: (0, 0)),
            pl.BlockSpec((NH * QD, 1), lambda i: (0, 0)),
            pl.BlockSpec((1, SIDE, SUB), lambda i: (i // TPG, 0, 0)),
            pl.BlockSpec((1, SIDE, SUB), lambda i: (i // TPG, 0, 0)),
        ],
        out_specs=[
            pl.BlockSpec((1, NH * K, TILE), lambda i: (i // TPG, 0, i % TPG)),
            pl.BlockSpec((1, NH * K, TILE), lambda i: (i // TPG, 0, i % TPG)),
        ],
        out_shape=[
            jax.ShapeDtypeStruct((NH, NH * K, GROUP), jnp.float32),
            jax.ShapeDtypeStruct((NH, NH * K, GROUP), jnp.int32),
        ],
    )(xt, wt, b2, k1, k2)
    sv = sv.transpose(0, 2, 1).reshape(NH, TOKENS, K).transpose(1, 0, 2)
    si = si.transpose(0, 2, 1).reshape(NH, TOKENS, K).transpose(1, 0, 2)
    return (si.reshape(BSZ, SEQ, NH, K), sv.reshape(BSZ, SEQ, NH, K))


# speedup vs baseline: 1.1594x; 1.1594x over previous
import jax
import jax.numpy as jnp
from jax import lax
from jax.experimental import pallas as pl

DIM = 1024
QD = 128
NH = 8
K = 8
SIDE = 1024
SUB = QD // 2
BSZ = 2
SEQ = 4096
TOKENS = BSZ * SEQ
GROUP = TOKENS // NH
TILE = 256
TPG = GROUP // TILE
ROWS = TILE * NH


def _topk8_ax0(s, n):
    iota = lax.broadcasted_iota(jnp.int32, s.shape, 0)
    neg_inf = jnp.float32(-jnp.inf)
    vals, idxs = [], []
    cur = s
    for k in range(K):
        m = jnp.max(cur, axis=0, keepdims=True)
        am = jnp.argmax(cur, axis=0, keepdims=True).astype(jnp.int32)
        vals.append(m)
        idxs.append(am)
        if k + 1 < K:
            cur = jnp.where(iota == am, neg_inf, cur)
    return jnp.concatenate(vals, axis=0), jnp.concatenate(idxs, axis=0)


def _router_kernel(xt_ref, wt_ref, b_ref, k1_ref, k2_ref, sv_ref, si_ref):
    q = jnp.dot(wt_ref[...], xt_ref[...], preferred_element_type=jnp.float32)
    q = q + b_ref[...]
    q1 = jnp.concatenate(
        [q[h * QD:h * QD + SUB, :] for h in range(NH)], axis=1)
    q2 = jnp.concatenate(
        [q[h * QD + SUB:(h + 1) * QD, :] for h in range(NH)], axis=1)
    s1 = jnp.dot(k1_ref[0], q1, preferred_element_type=jnp.float32)
    s2 = jnp.dot(k2_ref[0], q2, preferred_element_type=jnp.float32)
    v1, i1 = _topk8_ax0(s1, SIDE)
    v2, i2 = _topk8_ax0(s2, SIDE)
    c = jnp.concatenate([v1[j:j + 1, :] + v2 for j in range(K)], axis=0)
    cv, ci = _topk8_ax0(c, K * K)
    sub1 = ci // K
    sub2 = ci - sub1 * K
    a1 = jnp.zeros_like(ci)
    a2 = jnp.zeros_like(ci)
    for j in range(K):
        a1 = jnp.where(sub1 == j, i1[j:j + 1, :], a1)
        a2 = jnp.where(sub2 == j, i2[j:j + 1, :], a2)
    eid = a1 * SIDE + a2
    for h in range(NH):
        sv_ref[0, h * K:(h + 1) * K, :] = cv[:, h * TILE:(h + 1) * TILE]
        si_ref[0, h * K:(h + 1) * K, :] = eid[:, h * TILE:(h + 1) * TILE]


def kernel(x, W, b, keys):
    xt = x.reshape(TOKENS, DIM).T
    wt = W.T
    b2 = b.reshape(NH * QD, 1)
    k1 = keys[:, 0]
    k2 = keys[:, 1]
    grid = (TOKENS // TILE,)
    sv, si = pl.pallas_call(
        _router_kernel,
        grid=grid,
        in_specs=[
            pl.BlockSpec((DIM, TILE), lambda i: (0, i)),
            pl.BlockSpec((NH * QD, DIM), lambda i: (0, 0)),
            pl.BlockSpec((NH * QD, 1), lambda i: (0, 0)),
            pl.BlockSpec((1, SIDE, SUB), lambda i: (i // TPG, 0, 0)),
            pl.BlockSpec((1, SIDE, SUB), lambda i: (i // TPG, 0, 0)),
        ],
        out_specs=[
            pl.BlockSpec((1, NH * K, TILE), lambda i: (i // TPG, 0, i % TPG)),
            pl.BlockSpec((1, NH * K, TILE), lambda i: (i // TPG, 0, i % TPG)),
        ],
        out_shape=[
            jax.ShapeDtypeStruct((NH, NH * K, GROUP), jnp.float32),
            jax.ShapeDtypeStruct((NH, NH * K, GROUP), jnp.int32),
        ],
    )(xt, wt, b2, k1, k2)
    sv = sv.transpose(0, 2, 1).reshape(NH, TOKENS, K).transpose(1, 0, 2)
    si = si.transpose(0, 2, 1).reshape(NH, TOKENS, K).transpose(1, 0, 2)
    return (si.reshape(BSZ, SEQ, NH, K), sv.reshape(BSZ, SEQ, NH, K))
